# baseline (speedup 1.0000x reference)
import functools

import jax
import jax.numpy as jnp
from jax.experimental import pallas as pl


def _smoothness_body(a_ref, z_ref, zi_ref, out_ref, *, inv_n):
    i = pl.program_id(0)
    a = a_ref[...]
    zi = zi_ref[...]
    y = jnp.dot(a, z_ref[...], preferred_element_type=jnp.float32)
    d = jnp.sum(a, axis=1)
    s = jnp.sum(zi * zi, axis=1)
    contrib = (jnp.sum(d * s) - jnp.sum(zi * y)) * inv_n

    contrib = jnp.reshape(contrib, (1, 1))

    @pl.when(i == 0)
    def _init():
        out_ref[...] = contrib

    @pl.when(i != 0)
    def _acc():
        out_ref[...] += contrib


@jax.jit
def kernel(z, coords, precomputed_adj):
    del coords
    n, dim = z.shape
    blk = 1024
    grid = (n // blk,)
    out = pl.pallas_call(
        functools.partial(_smoothness_body, inv_n=1.0 / n),
        grid=grid,
        in_specs=[
            pl.BlockSpec((blk, n), lambda i: (i, 0)),
            pl.BlockSpec((n, dim), lambda i: (0, 0)),
            pl.BlockSpec((blk, dim), lambda i: (i, 0)),
        ],
        out_specs=pl.BlockSpec((1, 1), lambda i: (0, 0)),
        out_shape=jax.ShapeDtypeStruct((1, 1), jnp.float32),
    )(precomputed_adj, z, z)
    return out[0, 0]


# speedup vs baseline: 1.0037x; 1.0037x over previous
import functools

import jax
import jax.numpy as jnp
from jax.experimental import pallas as pl


def _smoothness_body(a_ref, z_ref, zi_ref, out_ref, *, inv_n):
    i = pl.program_id(0)
    a = a_ref[...]
    zi = zi_ref[...]
    y = jnp.dot(a, z_ref[...], preferred_element_type=jnp.float32)
    d = jnp.sum(a, axis=1)
    s = jnp.sum(zi * zi, axis=1)
    contrib = (jnp.sum(d * s) - jnp.sum(zi * y)) * inv_n

    contrib = jnp.reshape(contrib, (1, 1))

    @pl.when(i == 0)
    def _init():
        out_ref[...] = contrib

    @pl.when(i != 0)
    def _acc():
        out_ref[...] += contrib


@jax.jit
def kernel(z, coords, precomputed_adj):
    del coords
    n, dim = z.shape
    blk = 256
    grid = (n // blk,)
    out = pl.pallas_call(
        functools.partial(_smoothness_body, inv_n=1.0 / n),
        grid=grid,
        in_specs=[
            pl.BlockSpec((blk, n), lambda i: (i, 0)),
            pl.BlockSpec((n, dim), lambda i: (0, 0)),
            pl.BlockSpec((blk, dim), lambda i: (i, 0)),
        ],
        out_specs=pl.BlockSpec((1, 1), lambda i: (0, 0)),
        out_shape=jax.ShapeDtypeStruct((1, 1), jnp.float32),
    )(precomputed_adj, z, z)
    return out[0, 0]


# speedup vs baseline: 1.2502x; 1.2455x over previous
import functools

import jax
import jax.numpy as jnp
from jax.experimental import pallas as pl


def _smoothness_body(a0_ref, a1_ref, z_ref, zi0_ref, zi1_ref, out_ref, *, inv_n):
    i = pl.program_id(0)
    zfull = z_ref[...]
    contrib = jnp.float32(0.0)
    for a_ref, zi_ref in ((a0_ref, zi0_ref), (a1_ref, zi1_ref)):
        a = a_ref[...]
        zi = zi_ref[...]
        y = jnp.dot(a, zfull, preferred_element_type=jnp.float32)
        d = jnp.sum(a, axis=1)
        s = jnp.sum(zi * zi, axis=1)
        contrib += jnp.sum(d * s) - jnp.sum(zi * y)
    contrib = jnp.reshape(contrib * inv_n, (1, 1))

    @pl.when(i == 0)
    def _init():
        out_ref[...] = contrib

    @pl.when(i != 0)
    def _acc():
        out_ref[...] += contrib


@jax.jit
def kernel(z, coords, precomputed_adj):
    del coords
    n, dim = z.shape
    blk = 256
    grid = (n // (2 * blk),)
    out = pl.pallas_call(
        functools.partial(_smoothness_body, inv_n=1.0 / n),
        grid=grid,
        in_specs=[
            pl.BlockSpec((blk, n), lambda i: (2 * i, 0)),
            pl.BlockSpec((blk, n), lambda i: (2 * i + 1, 0)),
            pl.BlockSpec((n, dim), lambda i: (0, 0)),
            pl.BlockSpec((blk, dim), lambda i: (2 * i, 0)),
            pl.BlockSpec((blk, dim), lambda i: (2 * i + 1, 0)),
        ],
        out_specs=pl.BlockSpec((1, 1), lambda i: (0, 0)),
        out_shape=jax.ShapeDtypeStruct((1, 1), jnp.float32),
    )(precomputed_adj, precomputed_adj, z, z, z)
    return out[0, 0]
